# baseline (speedup 1.0000x reference)
import functools

import jax
import jax.numpy as jnp
from jax import lax
from jax.experimental import pallas as pl
from jax.experimental.pallas import tpu as pltpu
from jax.experimental.pallas import tpu_sc as plsc

B = 16384
D = 32
NW = 32
BPW = B // NW
CH = BPW // 128
BLK = 2048
TS = 2048
GUS = 18
GBS = 15
GU = 1 << GUS
GB = 1 << GBS
OFFS = (0, 20, 33, 65, 89, 96)


def _tp_body(x0_ref, x1_ref, x2_ref, x3_ref, o_ref):
    i0 = lax.broadcasted_iota(jnp.int32, (D, D), 0)
    i1 = lax.broadcasted_iota(jnp.int32, (D, D), 1)
    ident = (i0 == i1).astype(jnp.float32)
    for m, x_ref in enumerate((x0_ref, x1_ref, x2_ref, x3_ref)):
        o_ref[:, m * D:(m + 1) * D] = lax.dot_general(
            x_ref[...], ident, (((0,), (0,)), ((), ())),
            precision=lax.Precision.HIGHEST,
            preferred_element_type=jnp.float32)


def _transpose_pack(tabT, nrows, gblk):
    last = (nrows - 1) // TS

    def spec(m):
        return pl.BlockSpec(
            (D, TS), lambda j, m=m: (0, jnp.minimum(m * gblk + j, last)))

    return pl.pallas_call(
        _tp_body,
        grid=(gblk,),
        in_specs=[spec(0), spec(1), spec(2), spec(3)],
        out_specs=pl.BlockSpec((TS, 4 * D), lambda j: (j, 0)),
        out_shape=jax.ShapeDtypeStruct((gblk * TS, 4 * D), jnp.float32),
    )(tabT, tabT, tabT, tabT)


def _sc_gather_body(tabu, tabb, uidx, bidx, gue_out, gbe_out,
                    riv, qiv, gv, sem):
    wid = lax.axis_index("s") * 2 + lax.axis_index("c")
    base = wid * BPW
    for idx_hbm, tab, out, g in ((uidx, tabu, gue_out, GU),
                                 (bidx, tabb, gbe_out, GB)):
        pltpu.sync_copy(idx_hbm.at[pl.ds(base, BPW)], riv)
        for c in range(CH):
            for k in range(8):
                v = riv[pl.ds(c * 128 + k * 16, 16)]
                qiv[c, pl.ds(k * 16, 16)] = v & (g - 1)
        copies = [pltpu.async_copy(tab.at[qiv.at[c]],
                                   gv.at[pl.ds(c * 128, 128)], sem)
                  for c in range(CH)]
        for cp in copies:
            cp.wait()
        pltpu.sync_copy(gv, out.at[pl.ds(base, BPW)])


def _sc_gather(tabu, tabb, uidx, bidx):
    mesh = plsc.VectorSubcoreMesh(core_axis_name="c", subcore_axis_name="s")
    return pl.kernel(
        _sc_gather_body,
        mesh=mesh,
        out_type=[jax.ShapeDtypeStruct((B, 128), jnp.float32),
                  jax.ShapeDtypeStruct((B, 128), jnp.float32)],
        scratch_types=[
            pltpu.VMEM((BPW,), jnp.int32),
            pltpu.VMEM((CH, 128), jnp.int32),
            pltpu.VMEM((BPW, 128), jnp.float32),
            pltpu.SemaphoreType.DMA,
        ],
    )(tabu, tabb, uidx, bidx)


def _chunk_select(g, rem):
    return jnp.where(rem == 0, g[:, 0:32],
                     jnp.where(rem == 1, g[:, 32:64],
                               jnp.where(rem == 2, g[:, 64:96], g[:, 96:128])))


def _tc_body(ut_ref, bt_ref, gue_ref, gbe_ref, ur_ref, br_ref, tfT_ref,
             wut_ref, wbt_ref, wpack_ref, ttab_ref, out_ref):
    f32 = jnp.float32

    def dot_t(a, b):
        return lax.dot_general(a, b, (((1,), (1,)), ((), ())),
                               precision=lax.Precision.HIGHEST,
                               preferred_element_type=f32)

    def dot_n(a, b):
        return lax.dot_general(a, b, (((1,), (0,)), ((), ())),
                               precision=lax.Precision.HIGHEST,
                               preferred_element_type=f32)

    up = dot_t(ut_ref[...], wut_ref[...]) + wpack_ref[7:8, :]
    bp = dot_t(bt_ref[...], wbt_ref[...]) + wpack_ref[8:9, :]
    ue = _chunk_select(gue_ref[...], lax.shift_right_logical(ur_ref[...], GUS))
    be = _chunk_select(gbe_ref[...], lax.shift_right_logical(br_ref[...], GBS))
    wm1 = wpack_ref[0:1, :]
    wm2 = wpack_ref[1:2, :]
    wu1 = wpack_ref[2:3, :]
    wu2 = wpack_ref[3:4, :]
    wb1 = wpack_ref[4:5, :]
    wb2 = wpack_ref[5:6, :]
    t = (ue * (be * wm1 + wu1) + up * (bp * wm2 + wu2)
         + be * wb1 + bp * wb2)
    s = dot_t(jnp.ones((1, D), f32), t)

    tvec = dot_t(wpack_ref[6:7, :], ttab_ref[...])
    subl = lax.broadcasted_iota(jnp.int32, (128, BLK), 0)
    acc = jnp.zeros((128, BLK), f32)
    for j, off in enumerate(OFFS):
        row = tfT_ref[j:j + 1, :] + off
        acc = acc + (subl == row).astype(f32)
    ts = dot_n(tvec, acc)

    out_ref[...] = s + ts + wpack_ref[9:10, 0:1]


def _tc_combine(ut, bt, gue, gbe, ur, br, tfT, wut, wbt, wpack, ttab):
    grid = B // BLK
    return pl.pallas_call(
        _tc_body,
        grid=(grid,),
        in_specs=[
            pl.BlockSpec((BLK, 128), lambda i: (i, 0)),
            pl.BlockSpec((BLK, 128), lambda i: (i, 0)),
            pl.BlockSpec((BLK, 128), lambda i: (i, 0)),
            pl.BlockSpec((BLK, 128), lambda i: (i, 0)),
            pl.BlockSpec((BLK, 1), lambda i: (i, 0)),
            pl.BlockSpec((BLK, 1), lambda i: (i, 0)),
            pl.BlockSpec((6, BLK), lambda i: (0, i)),
            pl.BlockSpec((D, 128), lambda i: (0, 0)),
            pl.BlockSpec((D, 128), lambda i: (0, 0)),
            pl.BlockSpec((16, D), lambda i: (0, 0)),
            pl.BlockSpec((128, D), lambda i: (0, 0)),
        ],
        out_specs=pl.BlockSpec((1, BLK), lambda i: (0, i)),
        out_shape=jax.ShapeDtypeStruct((1, B), jnp.float32),
    )(ut, bt, gue, gbe, ur, br, tfT, wut, wbt, wpack, ttab)


def kernel(user, book, user_tag_embedding, book_tag_embedding, time_features,
           user_table, book_table, W_ut, b_ut, W_bt, b_bt,
           year_t, month_t, day_t, hour_t, weekday_t, isweekend_t,
           W_out, b_out):
    ui = user.astype(jnp.int32)
    bi = book.astype(jnp.int32)
    utab2 = _transpose_pack(user_table.T, 1000000, GU // TS)
    btab2 = _transpose_pack(book_table.T, 100000, GB // TS)
    gue, gbe = _sc_gather(utab2, btab2, ui, bi)

    w = W_out.reshape(224)
    wpack = jnp.zeros((16, D), jnp.float32)
    for r in range(7):
        wpack = wpack.at[r].set(w[r * 32:(r + 1) * 32])
    wpack = wpack.at[7].set(b_ut)
    wpack = wpack.at[8].set(b_bt)
    wpack = wpack.at[9, 0].set(b_out[0])

    ttab = jnp.zeros((128, D), jnp.float32)
    ttab = ttab.at[0:20, 0:10].set(year_t)
    ttab = ttab.at[20:33, 10:15].set(month_t)
    ttab = ttab.at[33:65, 15:20].set(day_t)
    ttab = ttab.at[65:89, 20:25].set(hour_t)
    ttab = ttab.at[89:96, 25:30].set(weekday_t)
    ttab = ttab.at[96:98, 30:32].set(isweekend_t)

    tfT = time_features.astype(jnp.int32).T
    out = _tc_combine(user_tag_embedding, book_tag_embedding, gue, gbe,
                      ui.reshape(B, 1), bi.reshape(B, 1), tfT,
                      W_ut, W_bt, wpack, ttab)
    return out.reshape(B)


# speedup vs baseline: 1.9198x; 1.9198x over previous
import functools

import jax
import jax.numpy as jnp
from jax import lax
from jax.experimental import pallas as pl
from jax.experimental.pallas import tpu as pltpu
from jax.experimental.pallas import tpu_sc as plsc

B = 16384
D = 32
NW = 32
BPW = B // NW
CH = BPW // 128
BLK = 2048
TS = 8192
GUS = 18
GBS = 15
GU = 1 << GUS
GB = 1 << GBS
OFFS = (0, 20, 33, 65, 89, 96)


def _tp_body(x0_ref, x1_ref, x2_ref, x3_ref, o_ref):
    o_ref[...] = jnp.concatenate(
        [jnp.transpose(x_ref[...], (1, 0))
         for x_ref in (x0_ref, x1_ref, x2_ref, x3_ref)], axis=1)


def _transpose_pack(tabT, nrows, gblk):
    last = (nrows - 1) // TS

    def spec(m):
        return pl.BlockSpec(
            (D, TS), lambda j, m=m: (0, jnp.minimum(m * gblk + j, last)))

    return pl.pallas_call(
        _tp_body,
        grid=(gblk,),
        in_specs=[spec(0), spec(1), spec(2), spec(3)],
        out_specs=pl.BlockSpec((TS, 4 * D), lambda j: (j, 0)),
        out_shape=jax.ShapeDtypeStruct((gblk * TS, 4 * D), jnp.float32),
    )(tabT, tabT, tabT, tabT)


def _sc_gather_body(tabu, tabb, uidx, bidx, gue_out, gbe_out,
                    riv, qiv, gv, sem):
    wid = lax.axis_index("s") * 2 + lax.axis_index("c")
    base = wid * BPW
    for idx_hbm, tab, out, g in ((uidx, tabu, gue_out, GU),
                                 (bidx, tabb, gbe_out, GB)):
        pltpu.sync_copy(idx_hbm.at[pl.ds(base, BPW)], riv)
        for c in range(CH):
            for k in range(8):
                v = riv[pl.ds(c * 128 + k * 16, 16)]
                qiv[c, pl.ds(k * 16, 16)] = v & (g - 1)
        copies = [pltpu.async_copy(tab.at[qiv.at[c]],
                                   gv.at[pl.ds(c * 128, 128)], sem)
                  for c in range(CH)]
        for cp in copies:
            cp.wait()
        pltpu.sync_copy(gv, out.at[pl.ds(base, BPW)])


def _sc_gather(tabu, tabb, uidx, bidx):
    mesh = plsc.VectorSubcoreMesh(core_axis_name="c", subcore_axis_name="s")
    return pl.kernel(
        _sc_gather_body,
        mesh=mesh,
        out_type=[jax.ShapeDtypeStruct((B, 128), jnp.float32),
                  jax.ShapeDtypeStruct((B, 128), jnp.float32)],
        scratch_types=[
            pltpu.VMEM((BPW,), jnp.int32),
            pltpu.VMEM((CH, 128), jnp.int32),
            pltpu.VMEM((BPW, 128), jnp.float32),
            pltpu.SemaphoreType.DMA,
        ],
    )(tabu, tabb, uidx, bidx)


def _chunk_select(g, rem):
    return jnp.where(rem == 0, g[:, 0:32],
                     jnp.where(rem == 1, g[:, 32:64],
                               jnp.where(rem == 2, g[:, 64:96], g[:, 96:128])))


def _tc_body(ut_ref, bt_ref, gue_ref, gbe_ref, ur_ref, br_ref, tfT_ref,
             wut_ref, wbt_ref, wpack_ref, ttab_ref, out_ref):
    f32 = jnp.float32

    def dot_t(a, b):
        return lax.dot_general(a, b, (((1,), (1,)), ((), ())),
                               precision=lax.Precision.HIGHEST,
                               preferred_element_type=f32)

    def dot_n(a, b):
        return lax.dot_general(a, b, (((1,), (0,)), ((), ())),
                               precision=lax.Precision.HIGHEST,
                               preferred_element_type=f32)

    up = dot_t(ut_ref[...], wut_ref[...]) + wpack_ref[7:8, :]
    bp = dot_t(bt_ref[...], wbt_ref[...]) + wpack_ref[8:9, :]
    ue = _chunk_select(gue_ref[...], lax.shift_right_logical(ur_ref[...], GUS))
    be = _chunk_select(gbe_ref[...], lax.shift_right_logical(br_ref[...], GBS))
    wm1 = wpack_ref[0:1, :]
    wm2 = wpack_ref[1:2, :]
    wu1 = wpack_ref[2:3, :]
    wu2 = wpack_ref[3:4, :]
    wb1 = wpack_ref[4:5, :]
    wb2 = wpack_ref[5:6, :]
    t = (ue * (be * wm1 + wu1) + up * (bp * wm2 + wu2)
         + be * wb1 + bp * wb2)
    s = dot_t(jnp.ones((1, D), f32), t)

    tvec = dot_t(wpack_ref[6:7, :], ttab_ref[...])
    subl = lax.broadcasted_iota(jnp.int32, (128, BLK), 0)
    acc = jnp.zeros((128, BLK), f32)
    for j, off in enumerate(OFFS):
        row = tfT_ref[j:j + 1, :] + off
        acc = acc + (subl == row).astype(f32)
    ts = dot_n(tvec, acc)

    out_ref[...] = s + ts + wpack_ref[9:10, 0:1]


def _tc_combine(ut, bt, gue, gbe, ur, br, tfT, wut, wbt, wpack, ttab):
    grid = B // BLK
    return pl.pallas_call(
        _tc_body,
        grid=(grid,),
        in_specs=[
            pl.BlockSpec((BLK, 128), lambda i: (i, 0)),
            pl.BlockSpec((BLK, 128), lambda i: (i, 0)),
            pl.BlockSpec((BLK, 128), lambda i: (i, 0)),
            pl.BlockSpec((BLK, 128), lambda i: (i, 0)),
            pl.BlockSpec((BLK, 1), lambda i: (i, 0)),
            pl.BlockSpec((BLK, 1), lambda i: (i, 0)),
            pl.BlockSpec((6, BLK), lambda i: (0, i)),
            pl.BlockSpec((D, 128), lambda i: (0, 0)),
            pl.BlockSpec((D, 128), lambda i: (0, 0)),
            pl.BlockSpec((16, D), lambda i: (0, 0)),
            pl.BlockSpec((128, D), lambda i: (0, 0)),
        ],
        out_specs=pl.BlockSpec((1, BLK), lambda i: (0, i)),
        out_shape=jax.ShapeDtypeStruct((1, B), jnp.float32),
    )(ut, bt, gue, gbe, ur, br, tfT, wut, wbt, wpack, ttab)


def kernel(user, book, user_tag_embedding, book_tag_embedding, time_features,
           user_table, book_table, W_ut, b_ut, W_bt, b_bt,
           year_t, month_t, day_t, hour_t, weekday_t, isweekend_t,
           W_out, b_out):
    ui = user.astype(jnp.int32)
    bi = book.astype(jnp.int32)
    utab2 = _transpose_pack(user_table.T, 1000000, GU // TS)
    btab2 = _transpose_pack(book_table.T, 100000, GB // TS)
    gue, gbe = _sc_gather(utab2, btab2, ui, bi)

    w = W_out.reshape(224)
    wpack = jnp.zeros((16, D), jnp.float32)
    for r in range(7):
        wpack = wpack.at[r].set(w[r * 32:(r + 1) * 32])
    wpack = wpack.at[7].set(b_ut)
    wpack = wpack.at[8].set(b_bt)
    wpack = wpack.at[9, 0].set(b_out[0])

    ttab = jnp.zeros((128, D), jnp.float32)
    ttab = ttab.at[0:20, 0:10].set(year_t)
    ttab = ttab.at[20:33, 10:15].set(month_t)
    ttab = ttab.at[33:65, 15:20].set(day_t)
    ttab = ttab.at[65:89, 20:25].set(hour_t)
    ttab = ttab.at[89:96, 25:30].set(weekday_t)
    ttab = ttab.at[96:98, 30:32].set(isweekend_t)

    tfT = time_features.astype(jnp.int32).T
    out = _tc_combine(user_tag_embedding, book_tag_embedding, gue, gbe,
                      ui.reshape(B, 1), bi.reshape(B, 1), tfT,
                      W_ut, W_bt, wpack, ttab)
    return out.reshape(B)


# speedup vs baseline: 1.9219x; 1.0011x over previous
import functools

import jax
import jax.numpy as jnp
from jax import lax
from jax.experimental import pallas as pl
from jax.experimental.pallas import tpu as pltpu
from jax.experimental.pallas import tpu_sc as plsc

B = 16384
D = 32
NW = 32
BPW = B // NW
CH = BPW // 128
BLK = 2048
TS = 8192
GUS = 18
GBS = 15
GU = 1 << GUS
GB = 1 << GBS
OFFS = (0, 20, 33, 65, 89, 96)


def _tp_body(x0_ref, x1_ref, x2_ref, x3_ref, o_ref):
    o_ref[...] = jnp.concatenate(
        [jnp.transpose(x_ref[...], (1, 0))
         for x_ref in (x0_ref, x1_ref, x2_ref, x3_ref)], axis=1)


def _transpose_pack(tabT, nrows, gblk):
    last = (nrows - 1) // TS

    def spec(m):
        return pl.BlockSpec(
            (D, TS), lambda j, m=m: (0, jnp.minimum(m * gblk + j, last)))

    return pl.pallas_call(
        _tp_body,
        grid=(gblk,),
        in_specs=[spec(0), spec(1), spec(2), spec(3)],
        out_specs=pl.BlockSpec((TS, 4 * D), lambda j: (j, 0)),
        out_shape=jax.ShapeDtypeStruct((gblk * TS, 4 * D), jnp.float32),
    )(tabT, tabT, tabT, tabT)


def _sc_gather_body(g, tab, idx_hbm, out, riv, qiv, gv, sem):
    wid = lax.axis_index("s") * 2 + lax.axis_index("c")
    base = wid * BPW
    pltpu.sync_copy(idx_hbm.at[pl.ds(base, BPW)], riv)
    for c in range(CH):
        for k in range(8):
            v = riv[pl.ds(c * 128 + k * 16, 16)]
            qiv[c, pl.ds(k * 16, 16)] = v & (g - 1)
    copies = [pltpu.async_copy(tab.at[qiv.at[c]],
                               gv.at[pl.ds(c * 128, 128)], sem)
              for c in range(CH)]
    for cp in copies:
        cp.wait()
    pltpu.sync_copy(gv, out.at[pl.ds(base, BPW)])


def _sc_gather(tab, idx, g):
    mesh = plsc.VectorSubcoreMesh(core_axis_name="c", subcore_axis_name="s")
    return pl.kernel(
        functools.partial(_sc_gather_body, g),
        mesh=mesh,
        out_type=jax.ShapeDtypeStruct((B, 128), jnp.float32),
        scratch_types=[
            pltpu.VMEM((BPW,), jnp.int32),
            pltpu.VMEM((CH, 128), jnp.int32),
            pltpu.VMEM((BPW, 128), jnp.float32),
            pltpu.SemaphoreType.DMA,
        ],
    )(tab, idx)


def _chunk_select(g, rem):
    return jnp.where(rem == 0, g[:, 0:32],
                     jnp.where(rem == 1, g[:, 32:64],
                               jnp.where(rem == 2, g[:, 64:96], g[:, 96:128])))


def _tc_body(ut_ref, bt_ref, gue_ref, gbe_ref, ur_ref, br_ref, tfT_ref,
             wut_ref, wbt_ref, wpack_ref, ttab_ref, out_ref):
    f32 = jnp.float32

    def dot_t(a, b):
        return lax.dot_general(a, b, (((1,), (1,)), ((), ())),
                               precision=lax.Precision.HIGHEST,
                               preferred_element_type=f32)

    def dot_n(a, b):
        return lax.dot_general(a, b, (((1,), (0,)), ((), ())),
                               precision=lax.Precision.HIGHEST,
                               preferred_element_type=f32)

    up = dot_t(ut_ref[...], wut_ref[...]) + wpack_ref[7:8, :]
    bp = dot_t(bt_ref[...], wbt_ref[...]) + wpack_ref[8:9, :]
    ue = _chunk_select(gue_ref[...], lax.shift_right_logical(ur_ref[...], GUS))
    be = _chunk_select(gbe_ref[...], lax.shift_right_logical(br_ref[...], GBS))
    wm1 = wpack_ref[0:1, :]
    wm2 = wpack_ref[1:2, :]
    wu1 = wpack_ref[2:3, :]
    wu2 = wpack_ref[3:4, :]
    wb1 = wpack_ref[4:5, :]
    wb2 = wpack_ref[5:6, :]
    t = (ue * (be * wm1 + wu1) + up * (bp * wm2 + wu2)
         + be * wb1 + bp * wb2)
    s = dot_t(jnp.ones((1, D), f32), t)

    tvec = dot_t(wpack_ref[6:7, :], ttab_ref[...])
    subl = lax.broadcasted_iota(jnp.int32, (128, BLK), 0)
    acc = jnp.zeros((128, BLK), f32)
    for j, off in enumerate(OFFS):
        row = tfT_ref[j:j + 1, :] + off
        acc = acc + (subl == row).astype(f32)
    ts = dot_n(tvec, acc)

    out_ref[...] = s + ts + wpack_ref[9:10, 0:1]


def _tc_combine(ut, bt, gue, gbe, ur, br, tfT, wut, wbt, wpack, ttab):
    grid = B // BLK
    return pl.pallas_call(
        _tc_body,
        grid=(grid,),
        in_specs=[
            pl.BlockSpec((BLK, 128), lambda i: (i, 0)),
            pl.BlockSpec((BLK, 128), lambda i: (i, 0)),
            pl.BlockSpec((BLK, 128), lambda i: (i, 0)),
            pl.BlockSpec((BLK, 128), lambda i: (i, 0)),
            pl.BlockSpec((BLK, 1), lambda i: (i, 0)),
            pl.BlockSpec((BLK, 1), lambda i: (i, 0)),
            pl.BlockSpec((6, BLK), lambda i: (0, i)),
            pl.BlockSpec((D, 128), lambda i: (0, 0)),
            pl.BlockSpec((D, 128), lambda i: (0, 0)),
            pl.BlockSpec((16, D), lambda i: (0, 0)),
            pl.BlockSpec((128, D), lambda i: (0, 0)),
        ],
        out_specs=pl.BlockSpec((1, BLK), lambda i: (0, i)),
        out_shape=jax.ShapeDtypeStruct((1, B), jnp.float32),
    )(ut, bt, gue, gbe, ur, br, tfT, wut, wbt, wpack, ttab)


def kernel(user, book, user_tag_embedding, book_tag_embedding, time_features,
           user_table, book_table, W_ut, b_ut, W_bt, b_bt,
           year_t, month_t, day_t, hour_t, weekday_t, isweekend_t,
           W_out, b_out):
    ui = user.astype(jnp.int32)
    bi = book.astype(jnp.int32)
    btab2 = _transpose_pack(book_table.T, 100000, GB // TS)
    gbe = _sc_gather(btab2, bi, GB)
    utab2 = _transpose_pack(user_table.T, 1000000, GU // TS)
    gue = _sc_gather(utab2, ui, GU)

    w = W_out.reshape(224)
    wpack = jnp.zeros((16, D), jnp.float32)
    for r in range(7):
        wpack = wpack.at[r].set(w[r * 32:(r + 1) * 32])
    wpack = wpack.at[7].set(b_ut)
    wpack = wpack.at[8].set(b_bt)
    wpack = wpack.at[9, 0].set(b_out[0])

    ttab = jnp.zeros((128, D), jnp.float32)
    ttab = ttab.at[0:20, 0:10].set(year_t)
    ttab = ttab.at[20:33, 10:15].set(month_t)
    ttab = ttab.at[33:65, 15:20].set(day_t)
    ttab = ttab.at[65:89, 20:25].set(hour_t)
    ttab = ttab.at[89:96, 25:30].set(weekday_t)
    ttab = ttab.at[96:98, 30:32].set(isweekend_t)

    tfT = time_features.astype(jnp.int32).T
    out = _tc_combine(user_tag_embedding, book_tag_embedding, gue, gbe,
                      ui.reshape(B, 1), bi.reshape(B, 1), tfT,
                      W_ut, W_bt, wpack, ttab)
    return out.reshape(B)
